# baseline (speedup 1.0000x reference)
import functools

import jax
import jax.numpy as jnp
from jax import lax
from jax.experimental import pallas as pl
from jax.experimental.pallas import tpu as pltpu
from jax.experimental.pallas import tpu_sc as plsc

V = 1000000
VP = V // 2
B = 16384
D = 64
P = 5
CTX = 25
NW = 32
CV = 256
HP = CV // 2
ROUNDS = 122
MAIN_V = NW * CV * ROUNDS
TAIL128_V = 999936
K = 16
NC = K * CTX
CHUNKS = B // NW // K


def _mesh():
  return plsc.VectorSubcoreMesh(core_axis_name="c", subcore_axis_name="s")


def _convert(ie_t, oe_t, ie_tail, oe_tail):
  @functools.partial(
      pl.kernel,
      mesh=_mesh(),
      out_type=(jax.ShapeDtypeStruct((VP, 128), jnp.float32),
                jax.ShapeDtypeStruct((VP, 128), jnp.float32)),
      compiler_params=pltpu.CompilerParams(use_tc_tiling_on_sc=True,
                                           needs_layout_passes=False),
      scratch_types=[
          pltpu.VMEM((D, CV), jnp.float32),
          pltpu.VMEM((D, CV), jnp.float32),
          pltpu.VMEM((HP, 128), jnp.float32),
          pltpu.VMEM((HP, 128), jnp.float32),
          pltpu.SemaphoreType.DMA,
          pltpu.SemaphoreType.DMA,
          pltpu.SemaphoreType.DMA,
          pltpu.SemaphoreType.DMA,
      ],
  )
  def conv(ie_hbm, oe_hbm, ie_tl, oe_tl, ie_c, oe_c,
           bin0, bin1, bout0, bout1, sin0, sin1, sout0, sout1):
    wid = lax.axis_index("c") * 16 + lax.axis_index("s")
    lane = lax.broadcasted_iota(jnp.int32, (16,), 0)
    bins = (bin0, bin1)
    bouts = (bout0, bout1)
    sins = (sin0, sin1)
    souts = (sout0, sout1)
    tbls = (ie_hbm, oe_hbm)
    outs = (ie_c, oe_c)

    def transpose_block(bi, npairs):
      def pair_body(p, carry):
        for half in range(2):
          vrel = 2 * p + half
          vsp = jnp.full((16,), 0, jnp.int32) + vrel
          for k in range(4):
            seg = plsc.load_gather(bins[bi], [lane + 16 * k, vsp])
            bouts[bi].at[p][pl.ds(half * 64 + k * 16, 16)] = seg
        return carry
      lax.fori_loop(0, npairs, pair_body, 0, unroll=False)

    def v0_of(r):
      return (r * NW + wid) * CV

    for ti in range(2):
      pltpu.async_copy(tbls[ti].at[:, pl.ds(v0_of(0), CV)], bins[ti], sins[ti])

    def round_body(r, carry):
      v0 = v0_of(r)
      p0 = pl.multiple_of(v0 // 2, HP)
      for ti in range(2):
        @pl.when(r > 0)
        def _drain():
          pltpu.make_async_copy(
              bouts[ti], outs[ti].at[pl.ds(p0, HP)], souts[ti]).wait()
        pltpu.make_async_copy(
            tbls[ti].at[:, pl.ds(v0, CV)], bins[ti], sins[ti]).wait()
        transpose_block(ti, HP)
        pltpu.async_copy(bouts[ti], outs[ti].at[pl.ds(p0, HP)], souts[ti])

        @pl.when(r + 1 < ROUNDS)
        def _next():
          pltpu.async_copy(
              tbls[ti].at[:, pl.ds(v0_of(r + 1), CV)], bins[ti], sins[ti])
      return carry

    lax.fori_loop(0, ROUNDS, round_body, 0, unroll=False)
    for ti in range(2):
      p_last = pl.multiple_of(v0_of(ROUNDS - 1) // 2, HP)
      pltpu.make_async_copy(
          bouts[ti], outs[ti].at[pl.ds(p_last, HP)], souts[ti]).wait()

    @pl.when(wid < 2)
    def _extra():
      v0 = MAIN_V + wid * CV
      p0 = pl.multiple_of(v0 // 2, HP)
      for ti in range(2):
        pltpu.async_copy(
            tbls[ti].at[:, pl.ds(v0, CV)], bins[ti], sins[ti]).wait()
        transpose_block(ti, HP)
        pltpu.async_copy(bouts[ti], outs[ti].at[pl.ds(p0, HP)], souts[ti]).wait()

    @pl.when(wid == 2)
    def _tail():
      tails = (ie_tl, oe_tl)
      for ti in range(2):
        pltpu.async_copy(
            tails[ti], bins[ti].at[:, pl.ds(0, 128)], sins[ti]).wait()
        transpose_block(ti, 32)
        pltpu.async_copy(
            bouts[ti].at[pl.ds(0, 32)],
            outs[ti].at[pl.ds(TAIL128_V // 2, 32)], souts[ti]).wait()

  return conv(ie_t, oe_t, ie_tail, oe_tail)


def _dots(cw, ctx, ie_c, oe_c):
  @functools.partial(
      pl.kernel,
      mesh=_mesh(),
      out_type=jax.ShapeDtypeStruct((B, 32), jnp.float32),
      compiler_params=pltpu.CompilerParams(use_tc_tiling_on_sc=True),
      scratch_types=[
          pltpu.VMEM((K + 16,), jnp.int32),
          pltpu.VMEM((NC + 16,), jnp.int32),
          pltpu.VMEM((K,), jnp.int32),
          pltpu.VMEM((NC,), jnp.int32),
          pltpu.VMEM((K, 128), jnp.float32),
          pltpu.VMEM((NC, 128), jnp.float32),
          pltpu.VMEM((K, 32), jnp.float32),
          pltpu.SemaphoreType.DMA,
      ],
  )
  def dotk(cw_hbm, ctx_hbm, iec, oec, out_hbm,
           cidx_v, ctxidx_v, cpidx_v, ctxpidx_v,
           cprows_v, uprows_v, dots_v, sem):
    wid = lax.axis_index("c") * 16 + lax.axis_index("s")
    lane = lax.broadcasted_iota(jnp.int32, (16,), 0)
    shuf = [lane ^ 8, lane ^ 4, lane ^ 2, lane ^ 1]

    def hsum(x):
      for sx in shuf:
        x = x + x.at[sx].get(mode="promise_in_bounds")
      return x

    def chunk_body(t, carry):
      base = wid * (B // NW) + t * K
      pltpu.sync_copy(cw_hbm.at[pl.ds(base, K)], cidx_v.at[pl.ds(0, K)])
      pltpu.sync_copy(ctx_hbm.at[pl.ds(base * CTX, NC)],
                      ctxidx_v.at[pl.ds(0, NC)])
      cpidx_v[pl.ds(0, 16)] = jnp.right_shift(cidx_v[pl.ds(0, 16)], 1)
      for i in range(NC // 16):
        ctxpidx_v[pl.ds(i * 16, 16)] = jnp.right_shift(
            ctxidx_v[pl.ds(i * 16, 16)], 1)
      cps = [pltpu.async_copy(iec.at[cpidx_v], cprows_v, sem)]
      for i0 in range(0, NC, 128):
        ln = min(128, NC - i0)
        cps.append(pltpu.async_copy(
            oec.at[ctxpidx_v.at[pl.ds(i0, ln)]],
            uprows_v.at[pl.ds(i0, ln)], sem))
      for c in cps:
        c.wait()

      def center_body(b, c2):
        pb = jnp.bitwise_and(cidx_v[pl.ds(b, 16)][0], 1) * 64
        cr = cprows_v.at[b]
        c = [cr[pl.ds(pb + q * 16, 16)] for q in range(4)]
        dv0 = jnp.zeros((16,), jnp.float32)
        dv1 = jnp.zeros((16,), jnp.float32)
        for j in range(CTX):
          pu = jnp.bitwise_and(ctxidx_v[pl.ds(b * CTX + j, 16)][0], 1) * 64
          ur = uprows_v.at[b * CTX + j]
          s = c[0] * ur[pl.ds(pu, 16)]
          for q in range(1, 4):
            s = s + c[q] * ur[pl.ds(pu + q * 16, 16)]
          tot = hsum(s)
          m = lane == (j % 16)
          if j < 16:
            dv0 = jnp.where(m, tot, dv0)
          else:
            dv1 = jnp.where(m, tot, dv1)
        dr = dots_v.at[b]
        dr[pl.ds(0, 16)] = dv0
        dr[pl.ds(16, 16)] = dv1
        return c2

      lax.fori_loop(0, K, center_body, 0, unroll=False)
      pltpu.sync_copy(dots_v, out_hbm.at[pl.ds(pl.multiple_of(base, K), K)])
      return carry

    lax.fori_loop(0, CHUNKS, chunk_body, 0, unroll=False)

  return dotk(cw, ctx, ie_c, oe_c)


def _tc_loss(dots):
  def body(x_ref, o_ref):
    x = x_ref[...]
    col = lax.broadcasted_iota(jnp.int32, x.shape, 1)
    signed = jnp.where(col < P, x, -x)
    y = jax.nn.log_sigmoid(signed)
    y = jnp.where(col < CTX, y, 0.0)
    o_ref[0, 0] = -jnp.sum(y) / B

  return pl.pallas_call(
      body,
      out_shape=jax.ShapeDtypeStruct((1, 1), jnp.float32),
      out_specs=pl.BlockSpec(memory_space=pltpu.SMEM),
  )(dots)


def kernel(center_words, pos_words, neg_words, in_embed, out_embed):
  cw = center_words.astype(jnp.int32)
  ctx = jnp.concatenate([pos_words.astype(jnp.int32),
                         neg_words.astype(jnp.int32)], axis=1).reshape(-1)
  ie_t = jnp.transpose(in_embed)
  oe_t = jnp.transpose(out_embed)
  ie_tail = jnp.pad(ie_t[:, TAIL128_V:], ((0, 0), (0, 64)))
  oe_tail = jnp.pad(oe_t[:, TAIL128_V:], ((0, 0), (0, 64)))
  ie_c, oe_c = _convert(ie_t, oe_t, ie_tail, oe_tail)
  dots = _dots(cw, ctx, ie_c, oe_c)
  return _tc_loss(dots)[0, 0]


# speedup vs baseline: 2.0225x; 2.0225x over previous
import functools

import jax
import jax.numpy as jnp
from jax import lax
from jax.experimental import pallas as pl
from jax.experimental.pallas import tpu as pltpu
from jax.experimental.pallas import tpu_sc as plsc

V = 1000000
B = 16384
D = 64
P = 5
CTX = 25
NW = 32
CB = 256
NBLK = 1954
VP = NBLK * CB
K = 16
NC = K * CTX
CHUNKS = B // NW // K


def _convert_tc(ie_t, oe_t):
  def body(a0, a1, b0, b1, oa, ob):
    oa[:, 0:64] = a0[...].T
    oa[:, 64:128] = a1[...].T
    ob[:, 0:64] = b0[...].T
    ob[:, 64:128] = b1[...].T

  last_blk = (V + CB - 1) // CB - 1
  half0 = pl.BlockSpec((D, CB), lambda i: (0, i))
  half1 = pl.BlockSpec((D, CB), lambda i: (0, jnp.minimum(i + NBLK, last_blk)))
  out_spec = pl.BlockSpec((CB, 128), lambda i: (i, 0))
  return pl.pallas_call(
      body,
      grid=(NBLK,),
      in_specs=[half0, half1, half0, half1],
      out_specs=(out_spec, out_spec),
      out_shape=(jax.ShapeDtypeStruct((VP, 128), jnp.float32),
                 jax.ShapeDtypeStruct((VP, 128), jnp.float32)),
  )(ie_t, ie_t, oe_t, oe_t)


def _dots(cw, ctx, ie_c, oe_c):
  mesh = plsc.VectorSubcoreMesh(core_axis_name="c", subcore_axis_name="s")

  @functools.partial(
      pl.kernel,
      mesh=mesh,
      out_type=jax.ShapeDtypeStruct((B, 32), jnp.float32),
      compiler_params=pltpu.CompilerParams(use_tc_tiling_on_sc=True),
      scratch_types=[
          pltpu.VMEM((K + 16,), jnp.int32),
          pltpu.VMEM((NC + 16,), jnp.int32),
          pltpu.VMEM((K,), jnp.int32),
          pltpu.VMEM((NC,), jnp.int32),
          pltpu.VMEM((K, 128), jnp.float32),
          pltpu.VMEM((NC, 128), jnp.float32),
          pltpu.VMEM((K, 32), jnp.float32),
          pltpu.SemaphoreType.DMA,
      ],
  )
  def dotk(cw_hbm, ctx_hbm, iec, oec, out_hbm,
           cidx_v, ctxidx_v, cpidx_v, ctxpidx_v,
           cprows_v, uprows_v, dots_v, sem):
    wid = lax.axis_index("c") * 16 + lax.axis_index("s")
    lane = lax.broadcasted_iota(jnp.int32, (16,), 0)
    shuf = [lane ^ 8, lane ^ 4, lane ^ 2, lane ^ 1]

    def hsum(x):
      for sx in shuf:
        x = x + x.at[sx].get(mode="promise_in_bounds")
      return x

    def pair_row(idx16):
      return jnp.where(idx16 < VP, idx16, idx16 - VP)

    def chunk_body(t, carry):
      base = wid * (B // NW) + t * K
      pltpu.sync_copy(cw_hbm.at[pl.ds(base, K)], cidx_v.at[pl.ds(0, K)])
      pltpu.sync_copy(ctx_hbm.at[pl.ds(base * CTX, NC)],
                      ctxidx_v.at[pl.ds(0, NC)])
      cpidx_v[pl.ds(0, 16)] = pair_row(cidx_v[pl.ds(0, 16)])
      for i in range(NC // 16):
        ctxpidx_v[pl.ds(i * 16, 16)] = pair_row(ctxidx_v[pl.ds(i * 16, 16)])
      cps = [pltpu.async_copy(iec.at[cpidx_v], cprows_v, sem)]
      for i0 in range(0, NC, 128):
        ln = min(128, NC - i0)
        cps.append(pltpu.async_copy(
            oec.at[ctxpidx_v.at[pl.ds(i0, ln)]],
            uprows_v.at[pl.ds(i0, ln)], sem))
      for c in cps:
        c.wait()

      def center_body(b, c2):
        pb = jnp.where(cidx_v[pl.ds(b, 16)][0] < VP, 0, 64)
        cr = cprows_v.at[b]
        c = [cr[pl.ds(pb + q * 16, 16)] for q in range(4)]
        dv0 = jnp.zeros((16,), jnp.float32)
        dv1 = jnp.zeros((16,), jnp.float32)
        for j in range(CTX):
          pu = jnp.where(ctxidx_v[pl.ds(b * CTX + j, 16)][0] < VP, 0, 64)
          ur = uprows_v.at[b * CTX + j]
          s = c[0] * ur[pl.ds(pu, 16)]
          for q in range(1, 4):
            s = s + c[q] * ur[pl.ds(pu + q * 16, 16)]
          tot = hsum(s)
          m = lane == (j % 16)
          if j < 16:
            dv0 = jnp.where(m, tot, dv0)
          else:
            dv1 = jnp.where(m, tot, dv1)
        dr = dots_v.at[b]
        dr[pl.ds(0, 16)] = dv0
        dr[pl.ds(16, 16)] = dv1
        return c2

      lax.fori_loop(0, K, center_body, 0, unroll=False)
      pltpu.sync_copy(dots_v, out_hbm.at[pl.ds(pl.multiple_of(base, K), K)])
      return carry

    lax.fori_loop(0, CHUNKS, chunk_body, 0, unroll=False)

  return dotk(cw, ctx, ie_c, oe_c)


def _tc_loss(dots):
  def body(x_ref, o_ref):
    x = x_ref[...]
    col = lax.broadcasted_iota(jnp.int32, x.shape, 1)
    signed = jnp.where(col < P, x, -x)
    y = jax.nn.log_sigmoid(signed)
    y = jnp.where(col < CTX, y, 0.0)
    o_ref[0, 0] = -jnp.sum(y) / B

  return pl.pallas_call(
      body,
      out_shape=jax.ShapeDtypeStruct((1, 1), jnp.float32),
      out_specs=pl.BlockSpec(memory_space=pltpu.SMEM),
  )(dots)


def kernel(center_words, pos_words, neg_words, in_embed, out_embed):
  cw = center_words.astype(jnp.int32)
  ctx = jnp.concatenate([pos_words.astype(jnp.int32),
                         neg_words.astype(jnp.int32)], axis=1).reshape(-1)
  ie_t = jnp.transpose(in_embed)
  oe_t = jnp.transpose(out_embed)
  ie_c, oe_c = _convert_tc(ie_t, oe_t)
  dots = _dots(cw, ctx, ie_c, oe_c)
  return _tc_loss(dots)[0, 0]


# speedup vs baseline: 3.7328x; 1.8457x over previous
import functools

import jax
import jax.numpy as jnp
from jax import lax
from jax.experimental import pallas as pl
from jax.experimental.pallas import tpu as pltpu
from jax.experimental.pallas import tpu_sc as plsc

V = 1000000
B = 16384
D = 64
P = 5
CTX = 25
NW = 32
CB = 1024
NBLK = 489
VP = NBLK * CB
K = 16
NC = K * CTX
CHUNKS = B // NW // K


def _convert_tc(ie_t, oe_t):
  def body(a0, a1, b0, b1, oa, ob):
    oa[:, 0:64] = a0[...].T
    oa[:, 64:128] = a1[...].T
    ob[:, 0:64] = b0[...].T
    ob[:, 64:128] = b1[...].T

  last_blk = (V + CB - 1) // CB - 1
  half0 = pl.BlockSpec((D, CB), lambda i: (0, i))
  half1 = pl.BlockSpec((D, CB), lambda i: (0, jnp.minimum(i + NBLK, last_blk)))
  out_spec = pl.BlockSpec((CB, 128), lambda i: (i, 0))
  return pl.pallas_call(
      body,
      grid=(NBLK,),
      in_specs=[half0, half1, half0, half1],
      out_specs=(out_spec, out_spec),
      out_shape=(jax.ShapeDtypeStruct((VP, 128), jnp.float32),
                 jax.ShapeDtypeStruct((VP, 128), jnp.float32)),
  )(ie_t, ie_t, oe_t, oe_t)


def _dots(cw, ctx, ie_c, oe_c):
  mesh = plsc.VectorSubcoreMesh(core_axis_name="c", subcore_axis_name="s")

  @functools.partial(
      pl.kernel,
      mesh=mesh,
      out_type=jax.ShapeDtypeStruct((B, 32), jnp.float32),
      compiler_params=pltpu.CompilerParams(use_tc_tiling_on_sc=True),
      scratch_types=[
          pltpu.VMEM((K + 16,), jnp.int32),
          pltpu.VMEM((NC + 16,), jnp.int32),
          pltpu.VMEM((K,), jnp.int32),
          pltpu.VMEM((NC,), jnp.int32),
          pltpu.VMEM((K, 128), jnp.float32),
          pltpu.VMEM((NC, 128), jnp.float32),
          pltpu.VMEM((K, 32), jnp.float32),
          pltpu.SemaphoreType.DMA,
      ],
  )
  def dotk(cw_hbm, ctx_hbm, iec, oec, out_hbm,
           cidx_v, ctxidx_v, cpidx_v, ctxpidx_v,
           cprows_v, uprows_v, dots_v, sem):
    wid = lax.axis_index("c") * 16 + lax.axis_index("s")
    lane = lax.broadcasted_iota(jnp.int32, (16,), 0)
    shuf = [lane ^ 8, lane ^ 4, lane ^ 2, lane ^ 1]

    def hsum(x):
      for sx in shuf:
        x = x + x.at[sx].get(mode="promise_in_bounds")
      return x

    def pair_row(idx16):
      return jnp.where(idx16 < VP, idx16, idx16 - VP)

    def chunk_body(t, carry):
      base = wid * (B // NW) + t * K
      pltpu.sync_copy(cw_hbm.at[pl.ds(base, K)], cidx_v.at[pl.ds(0, K)])
      pltpu.sync_copy(ctx_hbm.at[pl.ds(base * CTX, NC)],
                      ctxidx_v.at[pl.ds(0, NC)])
      cpidx_v[pl.ds(0, 16)] = pair_row(cidx_v[pl.ds(0, 16)])
      for i in range(NC // 16):
        ctxpidx_v[pl.ds(i * 16, 16)] = pair_row(ctxidx_v[pl.ds(i * 16, 16)])
      cps = [pltpu.async_copy(iec.at[cpidx_v], cprows_v, sem)]
      for i0 in range(0, NC, 128):
        ln = min(128, NC - i0)
        cps.append(pltpu.async_copy(
            oec.at[ctxpidx_v.at[pl.ds(i0, ln)]],
            uprows_v.at[pl.ds(i0, ln)], sem))
      for c in cps:
        c.wait()

      def center_body(b, c2):
        pb = jnp.where(cidx_v[pl.ds(b, 16)][0] < VP, 0, 64)
        cr = cprows_v.at[b]
        c = [cr[pl.ds(pb + q * 16, 16)] for q in range(4)]
        dv0 = jnp.zeros((16,), jnp.float32)
        dv1 = jnp.zeros((16,), jnp.float32)
        for j in range(CTX):
          pu = jnp.where(ctxidx_v[pl.ds(b * CTX + j, 16)][0] < VP, 0, 64)
          ur = uprows_v.at[b * CTX + j]
          s = c[0] * ur[pl.ds(pu, 16)]
          for q in range(1, 4):
            s = s + c[q] * ur[pl.ds(pu + q * 16, 16)]
          tot = hsum(s)
          m = lane == (j % 16)
          if j < 16:
            dv0 = jnp.where(m, tot, dv0)
          else:
            dv1 = jnp.where(m, tot, dv1)
        dr = dots_v.at[b]
        dr[pl.ds(0, 16)] = dv0
        dr[pl.ds(16, 16)] = dv1
        return c2

      lax.fori_loop(0, K, center_body, 0, unroll=False)
      pltpu.sync_copy(dots_v, out_hbm.at[pl.ds(pl.multiple_of(base, K), K)])
      return carry

    lax.fori_loop(0, CHUNKS, chunk_body, 0, unroll=False)

  return dotk(cw, ctx, ie_c, oe_c)


def _tc_loss(dots):
  def body(x_ref, o_ref):
    x = x_ref[...]
    col = lax.broadcasted_iota(jnp.int32, x.shape, 1)
    signed = jnp.where(col < P, x, -x)
    y = jax.nn.log_sigmoid(signed)
    y = jnp.where(col < CTX, y, 0.0)
    o_ref[0, 0] = -jnp.sum(y) / B

  return pl.pallas_call(
      body,
      out_shape=jax.ShapeDtypeStruct((1, 1), jnp.float32),
      out_specs=pl.BlockSpec(memory_space=pltpu.SMEM),
  )(dots)


def kernel(center_words, pos_words, neg_words, in_embed, out_embed):
  cw = center_words.astype(jnp.int32)
  ctx = jnp.concatenate([pos_words.astype(jnp.int32),
                         neg_words.astype(jnp.int32)], axis=1).reshape(-1)
  ie_t = jnp.transpose(in_embed)
  oe_t = jnp.transpose(out_embed)
  ie_c, oe_c = _convert_tc(ie_t, oe_t)
  dots = _dots(cw, ctx, ie_c, oe_c)
  return _tc_loss(dots)[0, 0]


# speedup vs baseline: 4.4807x; 1.2003x over previous
import functools

import jax
import jax.numpy as jnp
from jax import lax
from jax.experimental import pallas as pl
from jax.experimental.pallas import tpu as pltpu
from jax.experimental.pallas import tpu_sc as plsc

V = 1000000
B = 16384
D = 64
P = 5
CTX = 25
NW = 32
CB = 2048
NBLK = 245
VP = NBLK * CB
K = 16
NC = K * CTX
CHUNKS = B // NW // K


def _convert_tc(ie_t, oe_t):
  def body(a0, a1, b0, b1, oa, ob):
    oa[:, 0:64] = a0[...].T
    oa[:, 64:128] = a1[...].T
    ob[:, 0:64] = b0[...].T
    ob[:, 64:128] = b1[...].T

  last_blk = (V + CB - 1) // CB - 1
  half0 = pl.BlockSpec((D, CB), lambda i: (0, i))
  half1 = pl.BlockSpec((D, CB), lambda i: (0, jnp.minimum(i + NBLK, last_blk)))
  out_spec = pl.BlockSpec((CB, 128), lambda i: (i, 0))
  return pl.pallas_call(
      body,
      grid=(NBLK,),
      in_specs=[half0, half1, half0, half1],
      out_specs=(out_spec, out_spec),
      out_shape=(jax.ShapeDtypeStruct((VP, 128), jnp.float32),
                 jax.ShapeDtypeStruct((VP, 128), jnp.float32)),
  )(ie_t, ie_t, oe_t, oe_t)


def _dots(cw, ctx, ie_c, oe_c):
  mesh = plsc.VectorSubcoreMesh(core_axis_name="c", subcore_axis_name="s")

  @functools.partial(
      pl.kernel,
      mesh=mesh,
      out_type=jax.ShapeDtypeStruct((B, 32), jnp.float32),
      compiler_params=pltpu.CompilerParams(use_tc_tiling_on_sc=True),
      scratch_types=[
          pltpu.VMEM((K + 16,), jnp.int32),
          pltpu.VMEM((NC + 16,), jnp.int32),
          pltpu.VMEM((K,), jnp.int32),
          pltpu.VMEM((NC,), jnp.int32),
          pltpu.VMEM((K, 128), jnp.float32),
          pltpu.VMEM((NC, 128), jnp.float32),
          pltpu.VMEM((K, 32), jnp.float32),
          pltpu.SemaphoreType.DMA,
      ],
  )
  def dotk(cw_hbm, ctx_hbm, iec, oec, out_hbm,
           cidx_v, ctxidx_v, cpidx_v, ctxpidx_v,
           cprows_v, uprows_v, dots_v, sem):
    wid = lax.axis_index("c") * 16 + lax.axis_index("s")
    lane = lax.broadcasted_iota(jnp.int32, (16,), 0)
    shuf = [lane ^ 8, lane ^ 4, lane ^ 2, lane ^ 1]

    def hsum(x):
      for sx in shuf:
        x = x + x.at[sx].get(mode="promise_in_bounds")
      return x

    def pair_row(idx16):
      return jnp.where(idx16 < VP, idx16, idx16 - VP)

    def chunk_body(t, carry):
      base = wid * (B // NW) + t * K
      pltpu.sync_copy(cw_hbm.at[pl.ds(base, K)], cidx_v.at[pl.ds(0, K)])
      pltpu.sync_copy(ctx_hbm.at[pl.ds(base * CTX, NC)],
                      ctxidx_v.at[pl.ds(0, NC)])
      cpidx_v[pl.ds(0, 16)] = pair_row(cidx_v[pl.ds(0, 16)])
      for i in range(NC // 16):
        ctxpidx_v[pl.ds(i * 16, 16)] = pair_row(ctxidx_v[pl.ds(i * 16, 16)])
      cps = [pltpu.async_copy(iec.at[cpidx_v], cprows_v, sem)]
      for i0 in range(0, NC, 128):
        ln = min(128, NC - i0)
        cps.append(pltpu.async_copy(
            oec.at[ctxpidx_v.at[pl.ds(i0, ln)]],
            uprows_v.at[pl.ds(i0, ln)], sem))
      for c in cps:
        c.wait()

      def center_body(b, c2):
        pb = jnp.where(cidx_v[pl.ds(b, 16)][0] < VP, 0, 64)
        cr = cprows_v.at[b]
        c = [cr[pl.ds(pb + q * 16, 16)] for q in range(4)]
        dv0 = jnp.zeros((16,), jnp.float32)
        dv1 = jnp.zeros((16,), jnp.float32)
        for j in range(CTX):
          pu = jnp.where(ctxidx_v[pl.ds(b * CTX + j, 16)][0] < VP, 0, 64)
          ur = uprows_v.at[b * CTX + j]
          s = c[0] * ur[pl.ds(pu, 16)]
          for q in range(1, 4):
            s = s + c[q] * ur[pl.ds(pu + q * 16, 16)]
          tot = hsum(s)
          m = lane == (j % 16)
          if j < 16:
            dv0 = jnp.where(m, tot, dv0)
          else:
            dv1 = jnp.where(m, tot, dv1)
        dr = dots_v.at[b]
        dr[pl.ds(0, 16)] = dv0
        dr[pl.ds(16, 16)] = dv1
        return c2

      lax.fori_loop(0, K, center_body, 0, unroll=False)
      pltpu.sync_copy(dots_v, out_hbm.at[pl.ds(pl.multiple_of(base, K), K)])
      return carry

    lax.fori_loop(0, CHUNKS, chunk_body, 0, unroll=False)

  return dotk(cw, ctx, ie_c, oe_c)


def _tc_loss(dots):
  def body(x_ref, o_ref):
    x = x_ref[...]
    col = lax.broadcasted_iota(jnp.int32, x.shape, 1)
    signed = jnp.where(col < P, x, -x)
    y = jax.nn.log_sigmoid(signed)
    y = jnp.where(col < CTX, y, 0.0)
    o_ref[0, 0] = -jnp.sum(y) / B

  return pl.pallas_call(
      body,
      out_shape=jax.ShapeDtypeStruct((1, 1), jnp.float32),
      out_specs=pl.BlockSpec(memory_space=pltpu.SMEM),
  )(dots)


def kernel(center_words, pos_words, neg_words, in_embed, out_embed):
  cw = center_words.astype(jnp.int32)
  ctx = jnp.concatenate([pos_words.astype(jnp.int32),
                         neg_words.astype(jnp.int32)], axis=1).reshape(-1)
  ie_t = jnp.transpose(in_embed)
  oe_t = jnp.transpose(out_embed)
  ie_c, oe_c = _convert_tc(ie_t, oe_t)
  dots = _dots(cw, ctx, ie_c, oe_c)
  return _tc_loss(dots)[0, 0]


# speedup vs baseline: 5.0102x; 1.1182x over previous
import functools

import jax
import jax.numpy as jnp
from jax import lax
from jax.experimental import pallas as pl
from jax.experimental.pallas import tpu as pltpu
from jax.experimental.pallas import tpu_sc as plsc

V = 1000000
B = 16384
D = 64
P = 5
CTX = 25
NW = 32
CB = 4096
NBLK = 123
VP = NBLK * CB
K = 16
NC = K * CTX
CHUNKS = B // NW // K


def _convert_tc(ie_t, oe_t):
  def body(a0, a1, b0, b1, oa, ob):
    oa[:, 0:64] = a0[...].T
    oa[:, 64:128] = a1[...].T
    ob[:, 0:64] = b0[...].T
    ob[:, 64:128] = b1[...].T

  last_blk = (V + CB - 1) // CB - 1
  half0 = pl.BlockSpec((D, CB), lambda i: (0, i))
  half1 = pl.BlockSpec((D, CB), lambda i: (0, jnp.minimum(i + NBLK, last_blk)))
  out_spec = pl.BlockSpec((CB, 128), lambda i: (i, 0))
  return pl.pallas_call(
      body,
      grid=(NBLK,),
      in_specs=[half0, half1, half0, half1],
      out_specs=(out_spec, out_spec),
      out_shape=(jax.ShapeDtypeStruct((VP, 128), jnp.float32),
                 jax.ShapeDtypeStruct((VP, 128), jnp.float32)),
  )(ie_t, ie_t, oe_t, oe_t)


def _dots(cw, ctx, ie_c, oe_c):
  mesh = plsc.VectorSubcoreMesh(core_axis_name="c", subcore_axis_name="s")

  @functools.partial(
      pl.kernel,
      mesh=mesh,
      out_type=jax.ShapeDtypeStruct((B, 32), jnp.float32),
      compiler_params=pltpu.CompilerParams(use_tc_tiling_on_sc=True),
      scratch_types=[
          pltpu.VMEM((K + 16,), jnp.int32),
          pltpu.VMEM((NC + 16,), jnp.int32),
          pltpu.VMEM((K,), jnp.int32),
          pltpu.VMEM((NC,), jnp.int32),
          pltpu.VMEM((K, 128), jnp.float32),
          pltpu.VMEM((NC, 128), jnp.float32),
          pltpu.VMEM((K, 32), jnp.float32),
          pltpu.SemaphoreType.DMA,
      ],
  )
  def dotk(cw_hbm, ctx_hbm, iec, oec, out_hbm,
           cidx_v, ctxidx_v, cpidx_v, ctxpidx_v,
           cprows_v, uprows_v, dots_v, sem):
    wid = lax.axis_index("c") * 16 + lax.axis_index("s")
    lane = lax.broadcasted_iota(jnp.int32, (16,), 0)
    shuf = [lane ^ 8, lane ^ 4, lane ^ 2, lane ^ 1]

    def hsum(x):
      for sx in shuf:
        x = x + x.at[sx].get(mode="promise_in_bounds")
      return x

    def pair_row(idx16):
      return jnp.where(idx16 < VP, idx16, idx16 - VP)

    def chunk_body(t, carry):
      base = wid * (B // NW) + t * K
      pltpu.sync_copy(cw_hbm.at[pl.ds(base, K)], cidx_v.at[pl.ds(0, K)])
      pltpu.sync_copy(ctx_hbm.at[pl.ds(base * CTX, NC)],
                      ctxidx_v.at[pl.ds(0, NC)])
      cpidx_v[pl.ds(0, 16)] = pair_row(cidx_v[pl.ds(0, 16)])
      for i in range(NC // 16):
        ctxpidx_v[pl.ds(i * 16, 16)] = pair_row(ctxidx_v[pl.ds(i * 16, 16)])
      cps = [pltpu.async_copy(iec.at[cpidx_v], cprows_v, sem)]
      for i0 in range(0, NC, 128):
        ln = min(128, NC - i0)
        cps.append(pltpu.async_copy(
            oec.at[ctxpidx_v.at[pl.ds(i0, ln)]],
            uprows_v.at[pl.ds(i0, ln)], sem))
      for c in cps:
        c.wait()

      def center_body(b, c2):
        pb = jnp.where(cidx_v[pl.ds(b, 16)][0] < VP, 0, 64)
        cr = cprows_v.at[b]
        c = [cr[pl.ds(pb + q * 16, 16)] for q in range(4)]
        dv0 = jnp.zeros((16,), jnp.float32)
        dv1 = jnp.zeros((16,), jnp.float32)
        for j in range(CTX):
          pu = jnp.where(ctxidx_v[pl.ds(b * CTX + j, 16)][0] < VP, 0, 64)
          ur = uprows_v.at[b * CTX + j]
          s = c[0] * ur[pl.ds(pu, 16)]
          for q in range(1, 4):
            s = s + c[q] * ur[pl.ds(pu + q * 16, 16)]
          tot = hsum(s)
          m = lane == (j % 16)
          if j < 16:
            dv0 = jnp.where(m, tot, dv0)
          else:
            dv1 = jnp.where(m, tot, dv1)
        dr = dots_v.at[b]
        dr[pl.ds(0, 16)] = dv0
        dr[pl.ds(16, 16)] = dv1
        return c2

      lax.fori_loop(0, K, center_body, 0, unroll=False)
      pltpu.sync_copy(dots_v, out_hbm.at[pl.ds(pl.multiple_of(base, K), K)])
      return carry

    lax.fori_loop(0, CHUNKS, chunk_body, 0, unroll=False)

  return dotk(cw, ctx, ie_c, oe_c)


def _tc_loss(dots):
  def body(x_ref, o_ref):
    x = x_ref[...]
    col = lax.broadcasted_iota(jnp.int32, x.shape, 1)
    signed = jnp.where(col < P, x, -x)
    y = jax.nn.log_sigmoid(signed)
    y = jnp.where(col < CTX, y, 0.0)
    o_ref[0, 0] = -jnp.sum(y) / B

  return pl.pallas_call(
      body,
      out_shape=jax.ShapeDtypeStruct((1, 1), jnp.float32),
      out_specs=pl.BlockSpec(memory_space=pltpu.SMEM),
  )(dots)


def kernel(center_words, pos_words, neg_words, in_embed, out_embed):
  cw = center_words.astype(jnp.int32)
  ctx = jnp.concatenate([pos_words.astype(jnp.int32),
                         neg_words.astype(jnp.int32)], axis=1).reshape(-1)
  ie_t = jnp.transpose(in_embed)
  oe_t = jnp.transpose(out_embed)
  ie_c, oe_c = _convert_tc(ie_t, oe_t)
  dots = _dots(cw, ctx, ie_c, oe_c)
  return _tc_loss(dots)[0, 0]


# speedup vs baseline: 5.0578x; 1.0095x over previous
import functools

import jax
import jax.numpy as jnp
from jax import lax
from jax.experimental import pallas as pl
from jax.experimental.pallas import tpu as pltpu
from jax.experimental.pallas import tpu_sc as plsc

V = 1000000
B = 16384
D = 64
P = 5
CTX = 25
NW = 32
CB = 8192
NBLK = 62
VP = NBLK * CB
K = 16
NC = K * CTX
CHUNKS = B // NW // K


def _convert_tc(ie_t, oe_t):
  def body(a0, a1, b0, b1, oa, ob):
    oa[:, 0:64] = a0[...].T
    oa[:, 64:128] = a1[...].T
    ob[:, 0:64] = b0[...].T
    ob[:, 64:128] = b1[...].T

  last_blk = (V + CB - 1) // CB - 1
  half0 = pl.BlockSpec((D, CB), lambda i: (0, i))
  half1 = pl.BlockSpec((D, CB), lambda i: (0, jnp.minimum(i + NBLK, last_blk)))
  out_spec = pl.BlockSpec((CB, 128), lambda i: (i, 0))
  return pl.pallas_call(
      body,
      grid=(NBLK,),
      in_specs=[half0, half1, half0, half1],
      out_specs=(out_spec, out_spec),
      out_shape=(jax.ShapeDtypeStruct((VP, 128), jnp.float32),
                 jax.ShapeDtypeStruct((VP, 128), jnp.float32)),
  )(ie_t, ie_t, oe_t, oe_t)


def _dots(cw, ctx, ie_c, oe_c):
  mesh = plsc.VectorSubcoreMesh(core_axis_name="c", subcore_axis_name="s")

  @functools.partial(
      pl.kernel,
      mesh=mesh,
      out_type=jax.ShapeDtypeStruct((B, 32), jnp.float32),
      compiler_params=pltpu.CompilerParams(use_tc_tiling_on_sc=True),
      scratch_types=[
          pltpu.VMEM((K + 16,), jnp.int32),
          pltpu.VMEM((NC + 16,), jnp.int32),
          pltpu.VMEM((K,), jnp.int32),
          pltpu.VMEM((NC,), jnp.int32),
          pltpu.VMEM((K, 128), jnp.float32),
          pltpu.VMEM((NC, 128), jnp.float32),
          pltpu.VMEM((K, 32), jnp.float32),
          pltpu.SemaphoreType.DMA,
      ],
  )
  def dotk(cw_hbm, ctx_hbm, iec, oec, out_hbm,
           cidx_v, ctxidx_v, cpidx_v, ctxpidx_v,
           cprows_v, uprows_v, dots_v, sem):
    wid = lax.axis_index("c") * 16 + lax.axis_index("s")
    lane = lax.broadcasted_iota(jnp.int32, (16,), 0)
    shuf = [lane ^ 8, lane ^ 4, lane ^ 2, lane ^ 1]

    def hsum(x):
      for sx in shuf:
        x = x + x.at[sx].get(mode="promise_in_bounds")
      return x

    def pair_row(idx16):
      return jnp.where(idx16 < VP, idx16, idx16 - VP)

    def chunk_body(t, carry):
      base = wid * (B // NW) + t * K
      pltpu.sync_copy(cw_hbm.at[pl.ds(base, K)], cidx_v.at[pl.ds(0, K)])
      pltpu.sync_copy(ctx_hbm.at[pl.ds(base * CTX, NC)],
                      ctxidx_v.at[pl.ds(0, NC)])
      cpidx_v[pl.ds(0, 16)] = pair_row(cidx_v[pl.ds(0, 16)])
      for i in range(NC // 16):
        ctxpidx_v[pl.ds(i * 16, 16)] = pair_row(ctxidx_v[pl.ds(i * 16, 16)])
      cps = [pltpu.async_copy(iec.at[cpidx_v], cprows_v, sem)]
      for i0 in range(0, NC, 128):
        ln = min(128, NC - i0)
        cps.append(pltpu.async_copy(
            oec.at[ctxpidx_v.at[pl.ds(i0, ln)]],
            uprows_v.at[pl.ds(i0, ln)], sem))
      for c in cps:
        c.wait()

      def center_body(b, c2):
        pb = jnp.where(cidx_v[pl.ds(b, 16)][0] < VP, 0, 64)
        cr = cprows_v.at[b]
        c = [cr[pl.ds(pb + q * 16, 16)] for q in range(4)]
        dv0 = jnp.zeros((16,), jnp.float32)
        dv1 = jnp.zeros((16,), jnp.float32)
        for j in range(CTX):
          pu = jnp.where(ctxidx_v[pl.ds(b * CTX + j, 16)][0] < VP, 0, 64)
          ur = uprows_v.at[b * CTX + j]
          s = c[0] * ur[pl.ds(pu, 16)]
          for q in range(1, 4):
            s = s + c[q] * ur[pl.ds(pu + q * 16, 16)]
          tot = hsum(s)
          m = lane == (j % 16)
          if j < 16:
            dv0 = jnp.where(m, tot, dv0)
          else:
            dv1 = jnp.where(m, tot, dv1)
        dr = dots_v.at[b]
        dr[pl.ds(0, 16)] = dv0
        dr[pl.ds(16, 16)] = dv1
        return c2

      lax.fori_loop(0, K, center_body, 0, unroll=False)
      pltpu.sync_copy(dots_v, out_hbm.at[pl.ds(pl.multiple_of(base, K), K)])
      return carry

    lax.fori_loop(0, CHUNKS, chunk_body, 0, unroll=False)

  return dotk(cw, ctx, ie_c, oe_c)


def _tc_loss(dots):
  def body(x_ref, o_ref):
    x = x_ref[...]
    col = lax.broadcasted_iota(jnp.int32, x.shape, 1)
    signed = jnp.where(col < P, x, -x)
    y = jax.nn.log_sigmoid(signed)
    y = jnp.where(col < CTX, y, 0.0)
    o_ref[0, 0] = -jnp.sum(y) / B

  return pl.pallas_call(
      body,
      out_shape=jax.ShapeDtypeStruct((1, 1), jnp.float32),
      out_specs=pl.BlockSpec(memory_space=pltpu.SMEM),
  )(dots)


def kernel(center_words, pos_words, neg_words, in_embed, out_embed):
  cw = center_words.astype(jnp.int32)
  ctx = jnp.concatenate([pos_words.astype(jnp.int32),
                         neg_words.astype(jnp.int32)], axis=1).reshape(-1)
  ie_t = jnp.transpose(in_embed)
  oe_t = jnp.transpose(out_embed)
  ie_c, oe_c = _convert_tc(ie_t, oe_t)
  dots = _dots(cw, ctx, ie_c, oe_c)
  return _tc_loss(dots)[0, 0]
